# baseline (speedup 1.0000x reference)
import functools

import jax
import jax.numpy as jnp
from jax import lax
from jax.experimental import pallas as pl
from jax.experimental.pallas import tpu as pltpu
from jax.experimental.pallas import tpu_sc as plsc

NUM_EMB = 1_000_000
DIM = 64
ROWS = 16384
COLS = 26
B = ROWS * COLS

NC = 2
NS = 16
NW = NC * NS

IDX_W = 128
CHUNK = 1024
SUB = CHUNK // IDX_W
B_PER_W = B // NW
N_CHUNKS = B_PER_W // CHUNK

assert B_PER_W % CHUNK == 0 and CHUNK % IDX_W == 0

_mesh = plsc.VectorSubcoreMesh(core_axis_name="c", subcore_axis_name="s")


@functools.partial(
    pl.kernel,
    mesh=_mesh,
    out_type=jax.ShapeDtypeStruct((B, DIM), jnp.float32),
    scratch_types=[
        pltpu.VMEM((SUB, IDX_W), jnp.int32),
        pltpu.VMEM((CHUNK, DIM), jnp.float32),
        pltpu.SemaphoreType.DMA,
        pltpu.SemaphoreType.DMA,
    ],
    compiler_params=pltpu.CompilerParams(use_tc_tiling_on_sc=False),
)
def _emb_lookup(idx_hbm, table_hbm, out_hbm, idx_v, rows_v, gsem, osem):
    wid = lax.axis_index("s") * NC + lax.axis_index("c")
    row0 = wid * (B_PER_W // IDX_W)

    def body(i, _):
        irow = row0 + i * SUB
        off = irow * IDX_W
        pltpu.sync_copy(idx_hbm.at[pl.ds(irow, SUB)], idx_v)
        copies = [
            pltpu.async_copy(
                table_hbm.at[idx_v.at[j]],
                rows_v.at[pl.ds(j * IDX_W, IDX_W)],
                gsem,
            )
            for j in range(SUB)
        ]
        for c in copies:
            c.wait()
        pltpu.async_copy(rows_v, out_hbm.at[pl.ds(off, CHUNK)], osem).wait()
        return 0

    lax.fori_loop(0, N_CHUNKS, body, 0)


def kernel(x, Weights):
    idx = x.reshape(B // IDX_W, IDX_W).astype(jnp.int32)
    out = _emb_lookup(idx, Weights)
    return out.reshape(ROWS, COLS, DIM)


# speedup vs baseline: 1.0075x; 1.0075x over previous
import functools

import jax
import jax.numpy as jnp
from jax import lax
from jax.experimental import pallas as pl
from jax.experimental.pallas import tpu as pltpu
from jax.experimental.pallas import tpu_sc as plsc

NUM_EMB = 1_000_000
DIM = 64
ROWS = 16384
COLS = 26
B = ROWS * COLS

NC = 2
NS = 16
NW = NC * NS

IDX_W = 128
CHUNK = 512
SUB = CHUNK // IDX_W
B_PER_W = B // NW
N_CHUNKS = B_PER_W // CHUNK
IDX_ROWS = B_PER_W // IDX_W

assert B_PER_W % CHUNK == 0 and CHUNK % IDX_W == 0 and N_CHUNKS % 2 == 0

_mesh = plsc.VectorSubcoreMesh(core_axis_name="c", subcore_axis_name="s")


@functools.partial(
    pl.kernel,
    mesh=_mesh,
    out_type=jax.ShapeDtypeStruct((B, DIM), jnp.float32),
    scratch_types=[
        pltpu.VMEM((IDX_ROWS, IDX_W), jnp.int32),
        pltpu.VMEM((CHUNK, DIM), jnp.float32),
        pltpu.VMEM((CHUNK, DIM), jnp.float32),
        pltpu.SemaphoreType.DMA,
        pltpu.SemaphoreType.DMA,
        pltpu.SemaphoreType.DMA,
    ],
    compiler_params=pltpu.CompilerParams(use_tc_tiling_on_sc=False),
)
def _emb_lookup(idx_hbm, table_hbm, out_hbm, idx_v, rows0, rows1, gsem,
                osem0, osem1):
    wid = lax.axis_index("s") * NC + lax.axis_index("c")
    row0 = wid * IDX_ROWS
    base = wid * B_PER_W
    rbufs = (rows0, rows1)
    osems = (osem0, osem1)

    def gather(i, rbuf):
        return [
            pltpu.async_copy(
                table_hbm.at[idx_v.at[i * SUB + j]],
                rbuf.at[pl.ds(j * IDX_W, IDX_W)],
                gsem,
            )
            for j in range(SUB)
        ]

    def wait_gather(i, rbuf):
        for c in gather_descr(rbuf):
            c.wait()

    def gather_descr(rbuf):
        return [
            pltpu.make_async_copy(
                table_hbm.at[idx_v.at[j]],
                rbuf.at[pl.ds(j * IDX_W, IDX_W)],
                gsem,
            )
            for j in range(SUB)
        ]

    def store(i, rbuf, osem):
        return pltpu.async_copy(
            rbuf, out_hbm.at[pl.ds(base + i * CHUNK, CHUNK)], osem)

    def wait_store(rbuf, osem):
        pltpu.make_async_copy(
            rbuf, out_hbm.at[pl.ds(base, CHUNK)], osem).wait()

    pltpu.sync_copy(idx_hbm.at[pl.ds(row0, IDX_ROWS)], idx_v)

    gather(0, rows0)
    wait_gather(0, rows0)
    gather(1, rows1)
    store(0, rows0, osem0)

    def body(k, _):
        i1 = 2 * k + 1
        wait_gather(i1, rows1)
        wait_store(rows0, osem0)
        gather(i1 + 1, rows0)
        store(i1, rows1, osem1)
        i2 = 2 * k + 2
        wait_gather(i2, rows0)
        wait_store(rows1, osem1)
        gather(i2 + 1, rows1)
        store(i2, rows0, osem0)
        return 0

    lax.fori_loop(0, N_CHUNKS // 2 - 1, body, 0)

    last = N_CHUNKS - 1
    wait_gather(last, rows1)
    wait_store(rows0, osem0)
    store(last, rows1, osem1)
    wait_store(rows1, osem1)


def kernel(x, Weights):
    idx = x.reshape(B // IDX_W, IDX_W).astype(jnp.int32)
    out = _emb_lookup(idx, Weights)
    return out.reshape(ROWS, COLS, DIM)
